# baseline (speedup 1.0000x reference)
import functools

import jax
import jax.numpy as jnp
from jax import lax
from jax.experimental import pallas as pl
from jax.experimental.pallas import tpu as pltpu
from jax.experimental.pallas import tpu_sc as plsc

EMB = 64
HID = 256
NC = 2
NS = 16
NW = NC * NS
CHUNK = 128


def _sc_gather_body(uids, iids, uemb, iemb, uout, iout,
                    uidx, iidx, urows, irows, sem, *, nchunk, bpw):
    wid = lax.axis_index("s") * NC + lax.axis_index("c")
    base = wid * bpw
    pltpu.sync_copy(uids.at[wid], uidx)
    pltpu.sync_copy(iids.at[wid], iidx)
    cps = []
    for j in range(nchunk):
        cps.append(pltpu.async_copy(
            uemb.at[uidx.at[j]], urows.at[pl.ds(j * CHUNK, CHUNK)], sem))
        cps.append(pltpu.async_copy(
            iemb.at[iidx.at[j]], irows.at[pl.ds(j * CHUNK, CHUNK)], sem))
    for cp in cps:
        cp.wait()
    pltpu.sync_copy(urows, uout.at[pl.ds(base, bpw)])
    pltpu.sync_copy(irows, iout.at[pl.ds(base, bpw)])


def _mlp_body(u, i, w1u, w1i, b1, w2t, b2, o):
    x = jnp.dot(u[...], w1u[...], preferred_element_type=jnp.float32)
    x = x + jnp.dot(i[...], w1i[...], preferred_element_type=jnp.float32)
    x = jnp.maximum(x + b1[...], 0.0)
    o[...] = jnp.sum(x * w2t[...], axis=1, keepdims=True) + b2[...]


def kernel(user_ids, item_ids, user_emb, item_emb, W1, b1, W2, b2):
    B = user_ids.shape[0]
    bpw = B // NW
    nchunk = bpw // CHUNK
    uids_r = user_ids.astype(jnp.int32).reshape(NW, nchunk, CHUNK)
    iids_r = item_ids.astype(jnp.int32).reshape(NW, nchunk, CHUNK)

    gather = pl.kernel(
        functools.partial(_sc_gather_body, nchunk=nchunk, bpw=bpw),
        out_type=(jax.ShapeDtypeStruct((B, EMB), jnp.float32),
                  jax.ShapeDtypeStruct((B, EMB), jnp.float32)),
        mesh=plsc.VectorSubcoreMesh(core_axis_name="c", subcore_axis_name="s"),
        scratch_types=[
            pltpu.VMEM((nchunk, CHUNK), jnp.int32),
            pltpu.VMEM((nchunk, CHUNK), jnp.int32),
            pltpu.VMEM((bpw, EMB), jnp.float32),
            pltpu.VMEM((bpw, EMB), jnp.float32),
            pltpu.SemaphoreType.DMA,
        ],
        compiler_params=pltpu.CompilerParams(use_tc_tiling_on_sc=False),
    )
    urows, irows = gather(uids_r, iids_r, user_emb, item_emb)

    BM = 2048
    out = pl.pallas_call(
        _mlp_body,
        grid=(B // BM,),
        in_specs=[
            pl.BlockSpec((BM, EMB), lambda i: (i, 0)),
            pl.BlockSpec((BM, EMB), lambda i: (i, 0)),
            pl.BlockSpec((EMB, HID), lambda i: (0, 0)),
            pl.BlockSpec((EMB, HID), lambda i: (0, 0)),
            pl.BlockSpec((1, HID), lambda i: (0, 0)),
            pl.BlockSpec((1, HID), lambda i: (0, 0)),
            pl.BlockSpec((1, 1), lambda i: (0, 0)),
        ],
        out_specs=pl.BlockSpec((BM, 1), lambda i: (i, 0)),
        out_shape=jax.ShapeDtypeStruct((B, 1), jnp.float32),
    )(urows, irows, W1[:EMB], W1[EMB:], b1.reshape(1, HID),
      W2.reshape(1, HID), b2.reshape(1, 1))
    return out
